# baseline (speedup 1.0000x reference)
import functools

import jax
import jax.numpy as jnp
from jax import lax
from jax.experimental import pallas as pl
from jax.experimental.pallas import tpu as pltpu
from jax.experimental.pallas import tpu_sc as plsc

B = 8
LQ = 32
LK = 2048
D = 64
MAX_REL = 128
NROW = 2 * MAX_REL + 1
NW = 32
Q = 256
STEPS_PER_PAIR = LK // Q
NSLOT = 2
BC = 4
NCALL = B // BC

_mesh = plsc.VectorSubcoreMesh(core_axis_name="c", subcore_axis_name="s")


def _make_sc_gather(bc):
    npair = bc * LQ
    ppw = npair // NW
    s_total = ppw * STEPS_PER_PAIR

    @functools.partial(
        pl.kernel,
        mesh=_mesh,
        compiler_params=pltpu.CompilerParams(needs_layout_passes=False),
        out_type=jax.ShapeDtypeStruct((bc, LQ, LK, D), jnp.float32),
        scratch_types=[
            pltpu.VMEM((NROW, D), jnp.float32),
            pltpu.VMEM((LK,), jnp.int32),
            pltpu.VMEM((npair + 16,), jnp.int32),
            pltpu.VMEM((NSLOT * Q, D), jnp.float32),
            pltpu.SemaphoreType.DMA,
            pltpu.SemaphoreType.DMA,
        ],
    )
    def _sc_gather(pq_hbm, pk_hbm, table_hbm, out_hbm, tab_v, pk_v, pq_v,
                   rows_v, ssem0, ssem1):
        wid = lax.axis_index("s") * 2 + lax.axis_index("c")
        bq = (wid * ppw) // LQ
        h1 = pltpu.async_copy(table_hbm, tab_v, ssem0)
        h2 = pltpu.async_copy(pk_hbm.at[bq], pk_v, ssem0)
        h3 = pltpu.async_copy(pq_hbm, pq_v.at[pl.ds(0, npair)], ssem0)
        h1.wait()
        h2.wait()
        h3.wait()
        lane = lax.iota(jnp.int32, 16)
        cvecs = [lane + c4 * 16 for c4 in range(D // 16)]
        _dnums = lax.GatherDimensionNumbers(
            offset_dims=(), collapsed_slice_dims=(0,), start_index_map=(0,)
        )

        def _splat_lane(vec, r):
            return lax.gather(
                vec, jnp.broadcast_to(r, (16, 1)).astype(jnp.int32),
                _dnums, (1,),
                mode=lax.GatherScatterMode.PROMISE_IN_BOUNDS,
            )

        def build_step(t, k):
            pq_scalar = pq_v[pl.ds(wid * ppw + t // STEPS_PER_PAIR, 16)][0]
            pq_splat = jnp.full((16,), pq_scalar, jnp.int32)
            jbase = (t % STEPS_PER_PAIR) * Q

            def grp_body(g, carry):
                pk16 = pk_v[pl.ds(jbase + g * 16, 16)]
                rows16 = jnp.clip(pk16 - pq_splat, -MAX_REL, MAX_REL) + MAX_REL
                dst_g = k * Q + g * 16

                @plsc.parallel_loop(0, 16, 1, unroll=8)
                def _rows(r):
                    rowsplat = _splat_lane(rows16, r)
                    for c4 in range(D // 16):
                        vals = plsc.load_gather(tab_v, [rowsplat, cvecs[c4]])
                        rows_v[dst_g + r, pl.ds(c4 * 16, 16)] = vals

                return carry

            lax.fori_loop(0, Q // 16, grp_body, 0)

        def fire_scatter(t, k, sem):
            pair = wid * ppw + t // STEPS_PER_PAIR
            iq = pair % LQ
            jb = (t % STEPS_PER_PAIR) * Q
            pltpu.async_copy(
                rows_v.at[pl.ds(k * Q, Q)],
                out_hbm.at[bq, iq, pl.ds(jb, Q), :],
                sem,
            )

        def wait_scatter(k, sem):
            pltpu.make_async_copy(
                rows_v.at[pl.ds(k * Q, Q)],
                out_hbm.at[0, 0, pl.ds(0, Q), :],
                sem,
            ).wait()

        build_step(0, 0)
        fire_scatter(0, 0, ssem0)
        build_step(1, 1)
        fire_scatter(1, 1, ssem1)

        def outer(o, carry):
            s0 = o * 2
            wait_scatter(0, ssem0)
            build_step(s0, 0)
            fire_scatter(s0, 0, ssem0)
            wait_scatter(1, ssem1)
            build_step(s0 + 1, 1)
            fire_scatter(s0 + 1, 1, ssem1)
            return carry

        lax.fori_loop(1, s_total // 2, outer, 0)
        wait_scatter(0, ssem0)
        wait_scatter(1, ssem1)

    return _sc_gather


_sc_gather_bc = _make_sc_gather(BC)


def kernel(position_q, position_k, embeddings_table):
    pq = position_q.astype(jnp.int32)
    pk = position_k.astype(jnp.int32)
    out = jnp.zeros((B, LQ, LK, D), jnp.float32)
    for c in range(NCALL):
        sl = slice(c * BC, (c + 1) * BC)
        part = _sc_gather_bc(pq[sl].reshape(BC * LQ), pk[sl], embeddings_table)
        out = lax.dynamic_update_slice(out, part, (c * BC, 0, 0, 0))
    return out


# speedup vs baseline: 1.4553x; 1.4553x over previous
import functools

import jax
import jax.numpy as jnp
from jax import lax
from jax.experimental import pallas as pl
from jax.experimental.pallas import tpu as pltpu
from jax.experimental.pallas import tpu_sc as plsc

B = 8
LQ = 32
LK = 2048
D = 64
MAX_REL = 128
NROW = 2 * MAX_REL + 1
NPAIR = B * LQ
NW = 32
PAIRS_PER_W = NPAIR // NW
Q = 256
S = (PAIRS_PER_W * LK) // Q
STEPS_PER_PAIR = LK // Q
NSLOT = 2

_mesh = plsc.VectorSubcoreMesh(core_axis_name="c", subcore_axis_name="s")


@functools.partial(
    pl.kernel,
    mesh=_mesh,
    compiler_params=pltpu.CompilerParams(needs_layout_passes=False),
    out_type=jax.ShapeDtypeStruct((B, LQ, LK, D), jnp.float32),
    scratch_types=[
        pltpu.VMEM((NROW, D), jnp.float32),
        pltpu.VMEM((LK,), jnp.int32),
        pltpu.VMEM((NPAIR + 16,), jnp.int32),
        pltpu.VMEM((NSLOT * Q, D), jnp.float32),
        pltpu.SemaphoreType.DMA,
        pltpu.SemaphoreType.DMA,
    ],
)
def _sc_gather(pq_hbm, pk_hbm, table_hbm, out_hbm, tab_v, pk_v, pq_v,
               rows_v, ssem0, ssem1):
    wid = lax.axis_index("s") * 2 + lax.axis_index("c")
    bq = wid // (LQ // PAIRS_PER_W)
    h1 = pltpu.async_copy(table_hbm, tab_v, ssem0)
    h2 = pltpu.async_copy(pk_hbm.at[bq], pk_v, ssem0)
    h3 = pltpu.async_copy(pq_hbm, pq_v.at[pl.ds(0, NPAIR)], ssem0)
    h1.wait()
    h2.wait()
    h3.wait()
    lane = lax.iota(jnp.int32, 16)
    cvecs = [lane + c4 * 16 for c4 in range(D // 16)]
    _dnums = lax.GatherDimensionNumbers(
        offset_dims=(), collapsed_slice_dims=(0,), start_index_map=(0,)
    )

    def _splat_lane(vec, r):
        return lax.gather(
            vec, jnp.broadcast_to(r, (16, 1)).astype(jnp.int32), _dnums, (1,),
            mode=lax.GatherScatterMode.PROMISE_IN_BOUNDS,
        )

    def build_step(t, k):
        pq_scalar = pq_v[pl.ds(wid * PAIRS_PER_W + t // STEPS_PER_PAIR, 16)][0]
        pq_splat = jnp.full((16,), pq_scalar, jnp.int32)
        jbase = (t % STEPS_PER_PAIR) * Q

        def grp_body(g, carry):
            pk16 = pk_v[pl.ds(jbase + g * 16, 16)]
            rows16 = jnp.clip(pk16 - pq_splat, -MAX_REL, MAX_REL) + MAX_REL
            dst_g = k * Q + g * 16

            @plsc.parallel_loop(0, 16, 1, unroll=8)
            def _rows(r):
                rowsplat = _splat_lane(rows16, r)
                for c4 in range(D // 16):
                    vals = plsc.load_gather(tab_v, [rowsplat, cvecs[c4]])
                    rows_v[dst_g + r, pl.ds(c4 * 16, 16)] = vals

            return carry

        lax.fori_loop(0, Q // 16, grp_body, 0)

    def fire_scatter(t, k, sem):
        iq = (wid % (LQ // PAIRS_PER_W)) * PAIRS_PER_W + t // STEPS_PER_PAIR
        jb = (t % STEPS_PER_PAIR) * Q
        pltpu.async_copy(
            rows_v.at[pl.ds(k * Q, Q)],
            out_hbm.at[bq, iq, pl.ds(jb, Q), :],
            sem,
        )

    def wait_scatter(k, sem):
        pltpu.make_async_copy(
            rows_v.at[pl.ds(k * Q, Q)],
            out_hbm.at[0, 0, pl.ds(0, Q), :],
            sem,
        ).wait()

    build_step(0, 0)
    fire_scatter(0, 0, ssem0)
    build_step(1, 1)
    fire_scatter(1, 1, ssem1)

    def outer(o, carry):
        s0 = o * 2
        wait_scatter(0, ssem0)
        build_step(s0, 0)
        fire_scatter(s0, 0, ssem0)
        wait_scatter(1, ssem1)
        build_step(s0 + 1, 1)
        fire_scatter(s0 + 1, 1, ssem1)
        return carry

    lax.fori_loop(1, S // 2, outer, 0)
    wait_scatter(0, ssem0)
    wait_scatter(1, ssem1)


def kernel(position_q, position_k, embeddings_table):
    pq = position_q.astype(jnp.int32).reshape(NPAIR)
    pk = position_k.astype(jnp.int32)
    return _sc_gather(pq, pk, embeddings_table)


# speedup vs baseline: 1.4557x; 1.0003x over previous
import functools

import jax
import jax.numpy as jnp
from jax import lax
from jax.experimental import pallas as pl
from jax.experimental.pallas import tpu as pltpu
from jax.experimental.pallas import tpu_sc as plsc

B = 8
LQ = 32
LK = 2048
D = 64
MAX_REL = 128
NROW = 2 * MAX_REL + 1
NPAIR = B * LQ
NW = 32
PAIRS_PER_W = NPAIR // NW
Q = 256
S = (PAIRS_PER_W * LK) // Q
STEPS_PER_PAIR = LK // Q
NSLOT = 2

_mesh = plsc.VectorSubcoreMesh(core_axis_name="c", subcore_axis_name="s")


@functools.partial(
    pl.kernel,
    mesh=_mesh,
    compiler_params=pltpu.CompilerParams(needs_layout_passes=False),
    out_type=jax.ShapeDtypeStruct((B, LQ, LK, D), jnp.float32),
    scratch_types=[
        pltpu.VMEM((NROW, D), jnp.float32),
        pltpu.VMEM((LK,), jnp.int32),
        pltpu.VMEM((NPAIR + 16,), jnp.int32),
        pltpu.VMEM((NSLOT * Q, D), jnp.float32),
        pltpu.SemaphoreType.DMA,
        pltpu.SemaphoreType.DMA,
    ],
)
def _sc_gather(pq_hbm, pk_hbm, table_hbm, out_hbm, tab_v, pk_v, pq_v,
               rows_v, ssem0, ssem1):
    wid = lax.axis_index("s") * 2 + lax.axis_index("c")
    bq = wid // (LQ // PAIRS_PER_W)
    h1 = pltpu.async_copy(table_hbm, tab_v, ssem0)
    h2 = pltpu.async_copy(pk_hbm.at[bq], pk_v, ssem0)
    h3 = pltpu.async_copy(pq_hbm, pq_v.at[pl.ds(0, NPAIR)], ssem0)
    h1.wait()
    h2.wait()
    h3.wait()
    lane = lax.iota(jnp.int32, 16)
    cvecs = [lane + c4 * 16 for c4 in range(D // 16)]
    _dnums = lax.GatherDimensionNumbers(
        offset_dims=(), collapsed_slice_dims=(0,), start_index_map=(0,)
    )

    def _splat_lane(vec, r):
        return lax.gather(
            vec, jnp.broadcast_to(r, (16, 1)).astype(jnp.int32), _dnums, (1,),
            mode=lax.GatherScatterMode.PROMISE_IN_BOUNDS,
        )

    def build_step(t, k):
        pq_scalar = pq_v[pl.ds(wid * PAIRS_PER_W + t // STEPS_PER_PAIR, 16)][0]
        pq_splat = jnp.full((16,), pq_scalar, jnp.int32)
        jbase = (t % STEPS_PER_PAIR) * Q

        def grp_body(g, carry):
            pk16 = pk_v[pl.ds(jbase + g * 16, 16)]
            rows16 = jnp.clip(pk16 - pq_splat, -MAX_REL, MAX_REL) + MAX_REL
            dst_g = k * Q + g * 16

            @plsc.parallel_loop(0, 16, 1, unroll=16)
            def _rows(r):
                rowsplat = _splat_lane(rows16, r)
                for c4 in range(D // 16):
                    vals = plsc.load_gather(tab_v, [rowsplat, cvecs[c4]])
                    rows_v[dst_g + r, pl.ds(c4 * 16, 16)] = vals

            return carry

        lax.fori_loop(0, Q // 16, grp_body, 0)

    def fire_scatter(t, k, sem):
        iq = (wid % (LQ // PAIRS_PER_W)) * PAIRS_PER_W + t // STEPS_PER_PAIR
        jb = (t % STEPS_PER_PAIR) * Q
        pltpu.async_copy(
            rows_v.at[pl.ds(k * Q, Q)],
            out_hbm.at[bq, iq, pl.ds(jb, Q), :],
            sem,
        )

    def wait_scatter(k, sem):
        pltpu.make_async_copy(
            rows_v.at[pl.ds(k * Q, Q)],
            out_hbm.at[0, 0, pl.ds(0, Q), :],
            sem,
        ).wait()

    build_step(0, 0)
    fire_scatter(0, 0, ssem0)
    build_step(1, 1)
    fire_scatter(1, 1, ssem1)

    def outer(o, carry):
        s0 = o * 2
        wait_scatter(0, ssem0)
        build_step(s0, 0)
        fire_scatter(s0, 0, ssem0)
        wait_scatter(1, ssem1)
        build_step(s0 + 1, 1)
        fire_scatter(s0 + 1, 1, ssem1)
        return carry

    lax.fori_loop(1, S // 2, outer, 0)
    wait_scatter(0, ssem0)
    wait_scatter(1, ssem1)


def kernel(position_q, position_k, embeddings_table):
    pq = position_q.astype(jnp.int32).reshape(NPAIR)
    pk = position_k.astype(jnp.int32)
    return _sc_gather(pq, pk, embeddings_table)
